# baseline (speedup 1.0000x reference)
import functools

import jax
import jax.numpy as jnp
from jax import lax
from jax.experimental import pallas as pl
from jax.experimental.pallas import tpu as pltpu
from jax.experimental.pallas import tpu_sc as plsc

_N = 10000
_E = 320000
_D1 = 128
_D2 = 48
_NCLS = 40

_NC = 2
_NS = 16
_NW = _NC * _NS
_NPAD = 10240
_RPT = _NPAD // _NS
_EPT = _E // _NW
_KD = 2000
_K1 = 80
_K2 = 400
_BLK = 640
_GRID = 16

_mesh = plsc.VectorSubcoreMesh(core_axis_name="c", subcore_axis_name="s")


def _fill(ref, n, value):
    def body(i, carry):
        ref[pl.ds(i * 16, 16)] = jnp.full((16,), value, jnp.float32)
        return carry
    lax.fori_loop(0, n // 16, body, 0)


def _deg_body(src2_h, dst2_h, out_h, idx_v, ones_v, zed_v, dego_sp, degi_sp):
    cid = lax.axis_index("c")
    sid = lax.axis_index("s")
    wid = cid * _NS + sid
    r0 = sid * _RPT
    _fill(ones_v, _EPT, 1.0)
    _fill(zed_v, _RPT, 0.0)
    pltpu.sync_copy(zed_v, dego_sp.at[pl.ds(r0, _RPT)])
    pltpu.sync_copy(zed_v, degi_sp.at[pl.ds(r0, _RPT)])
    plsc.subcore_barrier()

    pltpu.sync_copy(src2_h.at[wid], idx_v)
    pltpu.sync_copy(ones_v, dego_sp.at[idx_v], add=True)
    pltpu.sync_copy(dst2_h.at[wid], idx_v)
    pltpu.sync_copy(ones_v, degi_sp.at[idx_v], add=True)
    plsc.subcore_barrier()
    pltpu.sync_copy(dego_sp.at[pl.ds(r0, _RPT)],
                    out_h.at[pl.ds(cid * _NPAD + r0, _RPT)])
    pltpu.sync_copy(degi_sp.at[pl.ds(r0, _RPT)],
                    out_h.at[pl.ds((2 + cid) * _NPAD + r0, _RPT)])


_deg_call = pl.kernel(
    _deg_body,
    mesh=_mesh,
    out_type=jax.ShapeDtypeStruct((4 * _NPAD,), jnp.float32),
    scratch_types=[
        pltpu.VMEM((_EPT,), jnp.int32),
        pltpu.VMEM((_EPT,), jnp.float32),
        pltpu.VMEM((_RPT,), jnp.float32),
        pltpu.VMEM_SHARED((_NPAD,), jnp.float32),
        pltpu.VMEM_SHARED((_NPAD,), jnp.float32),
    ],
    compiler_params=pltpu.CompilerParams(use_tc_tiling_on_sc=False),
)


def _seg_body(k, nch, d, h_h, src3_h, dst3_h, out_h,
              sidx_v, didx_v, rows0_v, rows1_v, acc_sp, g0, g1):
    cid = lax.axis_index("c")
    sid = lax.axis_index("s")
    wid = cid * _NS + sid
    r0 = sid * _RPT

    def zbody(i, carry):
        for dd in range(d // 16):
            rows0_v[i, pl.ds(dd * 16, 16)] = jnp.zeros((16,), jnp.float32)
        return carry
    lax.fori_loop(0, k, zbody, 0)
    off = 0
    while off < _RPT:
        step = min(k, _RPT - off)
        pltpu.sync_copy(rows0_v.at[pl.ds(0, step)],
                        acc_sp.at[pl.ds(r0 + off, step)])
        off += step

    pltpu.sync_copy(src3_h.at[wid], sidx_v)
    pltpu.sync_copy(dst3_h.at[wid], didx_v)
    plsc.subcore_barrier()

    dummy = h_h.at[pl.ds(0, k)]
    pltpu.async_copy(h_h.at[sidx_v.at[0]], rows0_v, g0)

    def body(i, carry):
        j0 = 2 * i
        pltpu.async_copy(h_h.at[sidx_v.at[j0 + 1]], rows1_v, g1)
        pltpu.make_async_copy(dummy, rows0_v, g0).wait()
        pltpu.sync_copy(rows0_v, acc_sp.at[didx_v.at[j0]], add=True)
        pltpu.async_copy(h_h.at[sidx_v.at[j0 + 2]], rows0_v, g0)
        pltpu.make_async_copy(dummy, rows1_v, g1).wait()
        pltpu.sync_copy(rows1_v, acc_sp.at[didx_v.at[j0 + 1]], add=True)
        return carry

    lax.fori_loop(0, (nch - 1) // 2, body, 0)
    pltpu.make_async_copy(dummy, rows0_v, g0).wait()
    pltpu.sync_copy(rows0_v, acc_sp.at[didx_v.at[nch - 1]], add=True)
    plsc.subcore_barrier()
    pltpu.sync_copy(acc_sp.at[pl.ds(r0, _RPT)],
                    out_h.at[pl.ds(cid * _NPAD + r0, _RPT)])


def _make_seg(d, k):
    nch = _EPT // k
    assert nch % 2 == 1 and k % 8 == 0
    return pl.kernel(
        functools.partial(_seg_body, k, nch, d),
        mesh=_mesh,
        out_type=jax.ShapeDtypeStruct((2 * _NPAD, d), jnp.float32),
        scratch_types=[
            pltpu.VMEM((nch, k), jnp.int32),
            pltpu.VMEM((nch, k), jnp.int32),
            pltpu.VMEM((k, d), jnp.float32),
            pltpu.VMEM((k, d), jnp.float32),
            pltpu.VMEM_SHARED((_NPAD, d), jnp.float32),
            pltpu.SemaphoreType.DMA,
            pltpu.SemaphoreType.DMA,
        ],
        compiler_params=pltpu.CompilerParams(use_tc_tiling_on_sc=False),
    )


_seg_d1 = _make_seg(_D1, _K1)
_seg_d2 = _make_seg(_D2, _K2)


def _h1_body(deg_ref, feat_ref, h1_ref):
    d = jnp.transpose(deg_ref[...])
    deg_out = d[:, 0:1] + d[:, 1:2]
    norm_src = lax.rsqrt(jnp.maximum(deg_out, 1.0))
    h1_ref[...] = feat_ref[...] * norm_src


def _mid_body(a0_ref, a1_ref, deg_ref, w1_ref, b1_ref, w2_ref, h2_ref):
    d = jnp.transpose(deg_ref[...])
    deg_out = d[:, 0:1] + d[:, 1:2]
    deg_in = d[:, 2:3] + d[:, 3:4]
    norm_src = lax.rsqrt(jnp.maximum(deg_out, 1.0))
    norm_dst = lax.rsqrt(jnp.maximum(deg_in, 1.0))
    agg = a0_ref[...] + a1_ref[...]
    x1 = jnp.dot(agg, w1_ref[...], preferred_element_type=jnp.float32)
    x1 = jnp.maximum(x1 * norm_dst + b1_ref[...][None, :], 0.0)
    h2_ref[...] = jnp.dot(x1 * norm_src, w2_ref[...],
                          preferred_element_type=jnp.float32)


def _fin_body(aggp_ref, deg_ref, b2_ref, out_ref):
    d = jnp.transpose(deg_ref[...])[:_N]
    deg_in = d[:, 2:3] + d[:, 3:4]
    norm_dst = lax.rsqrt(jnp.maximum(deg_in, 1.0))
    agg = aggp_ref[pl.ds(0, _N), :] + aggp_ref[pl.ds(_NPAD, _N), :]
    res = agg[:, :_NCLS] * norm_dst + b2_ref[...][None, :]
    out_ref[...] = jnp.transpose(res)


def kernel(feat, edge_index, W1, b1, W2, b2):
    src = edge_index[0].astype(jnp.int32)
    dst = edge_index[1].astype(jnp.int32)
    src1 = src.reshape(_NW, _EPT // _K1, _K1)
    dst1 = dst.reshape(_NW, _EPT // _K1, _K1)
    src2 = src.reshape(_NW, _EPT // _K2, _K2)
    dst2 = dst.reshape(_NW, _EPT // _K2, _K2)

    deg4 = _deg_call(src.reshape(_NW, _EPT), dst.reshape(_NW, _EPT))
    degt = deg4.reshape(4, _NPAD)

    blk = 1280
    h1 = pl.pallas_call(
        _h1_body,
        grid=(8,),
        in_specs=[pl.BlockSpec((4, blk), lambda i: (0, i)),
                  pl.BlockSpec((blk, _D1), lambda i: (i, 0))],
        out_specs=pl.BlockSpec((blk, _D1), lambda i: (i, 0)),
        out_shape=jax.ShapeDtypeStruct((_N, _D1), jnp.float32),
    )(degt, feat)

    aggp = _seg_d1(h1, src1, dst1)

    w2p = jnp.pad(W2, ((0, 0), (0, _D2 - _NCLS)))
    h2 = pl.pallas_call(
        _mid_body,
        grid=(8,),
        in_specs=[pl.BlockSpec((blk, _D1), lambda i: (i, 0)),
                  pl.BlockSpec((blk, _D1), lambda i: (i + 8, 0)),
                  pl.BlockSpec((4, blk), lambda i: (0, i)),
                  pl.BlockSpec((_D1, _D1), lambda i: (0, 0)),
                  pl.BlockSpec((_D1,), lambda i: (0,)),
                  pl.BlockSpec((_D1, _D2), lambda i: (0, 0))],
        out_specs=pl.BlockSpec((blk, _D2), lambda i: (i, 0)),
        out_shape=jax.ShapeDtypeStruct((_N, _D2), jnp.float32),
    )(aggp, aggp, degt, W1, b1, w2p)

    agg2p = _seg_d2(h2, src2, dst2)

    out_t = pl.pallas_call(
        _fin_body,
        out_shape=jax.ShapeDtypeStruct((_NCLS, _N), jnp.float32),
    )(agg2p, degt, b2)
    return jnp.transpose(out_t)


# speedup vs baseline: 1.0064x; 1.0064x over previous
import functools

import jax
import jax.numpy as jnp
from jax import lax
from jax.experimental import pallas as pl
from jax.experimental.pallas import tpu as pltpu
from jax.experimental.pallas import tpu_sc as plsc

_N = 10000
_E = 320000
_D1 = 128
_D2 = 48
_NCLS = 40

_NC = 2
_NS = 16
_NW = _NC * _NS
_NPAD = 10240
_RPT = _NPAD // _NS
_EPT = _E // _NW
_K1 = 80
_K2 = 400

_mesh = plsc.VectorSubcoreMesh(core_axis_name="c", subcore_axis_name="s")


def _fill(ref, n, value):
    def body(i, carry):
        ref[pl.ds(i * 16, 16)] = jnp.full((16,), value, jnp.float32)
        return carry
    lax.fori_loop(0, n // 16, body, 0)


def _deg_body(src2_h, dst2_h, out_h, idx_v, ones_v, zed_v, dego_sp, degi_sp):
    cid = lax.axis_index("c")
    sid = lax.axis_index("s")
    wid = cid * _NS + sid
    r0 = sid * _RPT
    _fill(ones_v, _EPT, 1.0)
    _fill(zed_v, _RPT, 0.0)
    pltpu.sync_copy(zed_v, dego_sp.at[pl.ds(r0, _RPT)])
    pltpu.sync_copy(zed_v, degi_sp.at[pl.ds(r0, _RPT)])
    plsc.subcore_barrier()

    pltpu.sync_copy(src2_h.at[wid], idx_v)
    pltpu.sync_copy(ones_v, dego_sp.at[idx_v], add=True)
    pltpu.sync_copy(dst2_h.at[wid], idx_v)
    pltpu.sync_copy(ones_v, degi_sp.at[idx_v], add=True)
    plsc.subcore_barrier()
    pltpu.sync_copy(dego_sp.at[pl.ds(r0, _RPT)],
                    out_h.at[pl.ds(cid * _NPAD + r0, _RPT)])
    pltpu.sync_copy(degi_sp.at[pl.ds(r0, _RPT)],
                    out_h.at[pl.ds((2 + cid) * _NPAD + r0, _RPT)])


_deg_call = pl.kernel(
    _deg_body,
    mesh=_mesh,
    out_type=jax.ShapeDtypeStruct((4 * _NPAD,), jnp.float32),
    scratch_types=[
        pltpu.VMEM((_EPT,), jnp.int32),
        pltpu.VMEM((_EPT,), jnp.float32),
        pltpu.VMEM((_RPT,), jnp.float32),
        pltpu.VMEM_SHARED((_NPAD,), jnp.float32),
        pltpu.VMEM_SHARED((_NPAD,), jnp.float32),
    ],
    compiler_params=pltpu.CompilerParams(use_tc_tiling_on_sc=False),
)


def _seg_body(k, nch, d, h_h, src3_h, dst3_h, out_h,
              sidx_v, didx_v, rows0_v, rows1_v, acc_sp, g0, g1):
    cid = lax.axis_index("c")
    sid = lax.axis_index("s")
    wid = cid * _NS + sid
    r0 = sid * _RPT

    def zbody(i, carry):
        for dd in range(d // 16):
            rows0_v[i, pl.ds(dd * 16, 16)] = jnp.zeros((16,), jnp.float32)
        return carry
    lax.fori_loop(0, k, zbody, 0)
    off = 0
    while off < _RPT:
        step = min(k, _RPT - off)
        pltpu.sync_copy(rows0_v.at[pl.ds(0, step)],
                        acc_sp.at[pl.ds(r0 + off, step)])
        off += step

    pltpu.sync_copy(src3_h.at[wid], sidx_v)
    pltpu.sync_copy(dst3_h.at[wid], didx_v)
    plsc.subcore_barrier()

    dummy = h_h.at[pl.ds(0, k)]
    pltpu.async_copy(h_h.at[sidx_v.at[0]], rows0_v, g0)

    def body(i, carry):
        j0 = 2 * i
        pltpu.async_copy(h_h.at[sidx_v.at[j0 + 1]], rows1_v, g1)
        pltpu.make_async_copy(dummy, rows0_v, g0).wait()
        pltpu.sync_copy(rows0_v, acc_sp.at[didx_v.at[j0]], add=True)
        pltpu.async_copy(h_h.at[sidx_v.at[j0 + 2]], rows0_v, g0)
        pltpu.make_async_copy(dummy, rows1_v, g1).wait()
        pltpu.sync_copy(rows1_v, acc_sp.at[didx_v.at[j0 + 1]], add=True)
        return carry

    lax.fori_loop(0, (nch - 1) // 2, body, 0)
    pltpu.make_async_copy(dummy, rows0_v, g0).wait()
    pltpu.sync_copy(rows0_v, acc_sp.at[didx_v.at[nch - 1]], add=True)
    plsc.subcore_barrier()
    pltpu.sync_copy(acc_sp.at[pl.ds(r0, _RPT)],
                    out_h.at[pl.ds(cid * _NPAD + r0, _RPT)])


def _make_seg(d, k):
    nch = _EPT // k
    assert nch % 2 == 1 and k % 8 == 0
    return pl.kernel(
        functools.partial(_seg_body, k, nch, d),
        mesh=_mesh,
        out_type=jax.ShapeDtypeStruct((2 * _NPAD, d), jnp.float32),
        scratch_types=[
            pltpu.VMEM((nch, k), jnp.int32),
            pltpu.VMEM((nch, k), jnp.int32),
            pltpu.VMEM((k, d), jnp.float32),
            pltpu.VMEM((k, d), jnp.float32),
            pltpu.VMEM_SHARED((_NPAD, d), jnp.float32),
            pltpu.SemaphoreType.DMA,
            pltpu.SemaphoreType.DMA,
        ],
        compiler_params=pltpu.CompilerParams(use_tc_tiling_on_sc=False),
    )


_seg_d1 = _make_seg(_D1, _K1)
_seg_d2 = _make_seg(_D2, _K2)


def _h1_body(deg_ref, feat_ref, h1_ref):
    d = jnp.transpose(deg_ref[...])[:_N]
    deg_out = d[:, 0:1] + d[:, 1:2]
    norm_src = lax.rsqrt(jnp.maximum(deg_out, 1.0))
    h1_ref[...] = feat_ref[...] * norm_src


def _mid_body(aggp_ref, deg_ref, w1_ref, b1_ref, w2_ref, h2_ref):
    d = jnp.transpose(deg_ref[...])[:_N]
    deg_out = d[:, 0:1] + d[:, 1:2]
    deg_in = d[:, 2:3] + d[:, 3:4]
    norm_src = lax.rsqrt(jnp.maximum(deg_out, 1.0))
    norm_dst = lax.rsqrt(jnp.maximum(deg_in, 1.0))
    agg = aggp_ref[pl.ds(0, _N), :] + aggp_ref[pl.ds(_NPAD, _N), :]
    x1 = jnp.dot(agg, w1_ref[...], preferred_element_type=jnp.float32)
    x1 = jnp.maximum(x1 * norm_dst + b1_ref[...][None, :], 0.0)
    h2_ref[...] = jnp.dot(x1 * norm_src, w2_ref[...],
                          preferred_element_type=jnp.float32)


def _fin_body(aggp_ref, deg_ref, b2_ref, out_ref):
    d = jnp.transpose(deg_ref[...])[:_N]
    deg_in = d[:, 2:3] + d[:, 3:4]
    norm_dst = lax.rsqrt(jnp.maximum(deg_in, 1.0))
    agg = aggp_ref[pl.ds(0, _N), :] + aggp_ref[pl.ds(_NPAD, _N), :]
    res = agg[:, :_NCLS] * norm_dst + b2_ref[...][None, :]
    out_ref[...] = jnp.transpose(res)


def kernel(feat, edge_index, W1, b1, W2, b2):
    src = edge_index[0].astype(jnp.int32)
    dst = edge_index[1].astype(jnp.int32)
    src1 = src.reshape(_NW, _EPT // _K1, _K1)
    dst1 = dst.reshape(_NW, _EPT // _K1, _K1)
    src2 = src.reshape(_NW, _EPT // _K2, _K2)
    dst2 = dst.reshape(_NW, _EPT // _K2, _K2)

    deg4 = _deg_call(src.reshape(_NW, _EPT), dst.reshape(_NW, _EPT))
    degt = deg4.reshape(4, _NPAD)

    h1 = pl.pallas_call(
        _h1_body,
        out_shape=jax.ShapeDtypeStruct((_N, _D1), jnp.float32),
    )(degt, feat)

    aggp = _seg_d1(h1, src1, dst1)

    w2p = jnp.pad(W2, ((0, 0), (0, _D2 - _NCLS)))
    h2 = pl.pallas_call(
        _mid_body,
        out_shape=jax.ShapeDtypeStruct((_N, _D2), jnp.float32),
    )(aggp, degt, W1, b1, w2p)

    agg2p = _seg_d2(h2, src2, dst2)

    out_t = pl.pallas_call(
        _fin_body,
        out_shape=jax.ShapeDtypeStruct((_NCLS, _N), jnp.float32),
    )(agg2p, degt, b2)
    return jnp.transpose(out_t)
